# baseline (speedup 1.0000x reference)
import functools
import math

import jax
import jax.numpy as jnp
from jax import lax
from jax.experimental import pallas as pl
from jax.experimental.pallas import tpu as pltpu
from jax.experimental.pallas import tpu_sc as plsc

NUM_CORES = 2
NUM_SUBCORES = 16
NUM_WORKERS = NUM_CORES * NUM_SUBCORES
CHUNK = 800
PARTS = 8


def _emb_call(n_part, V, D, scale, part):
    n_per_w = n_part // NUM_WORKERS
    n_chunks = n_per_w // CHUNK
    assert n_chunks * CHUNK * NUM_WORKERS == n_part
    mesh = plsc.VectorSubcoreMesh(core_axis_name="c", subcore_axis_name="s")

    @functools.partial(
        pl.kernel,
        mesh=mesh,
        compiler_params=pltpu.CompilerParams(use_tc_tiling_on_sc=False),
        out_type=jax.ShapeDtypeStruct((n_part, D), jnp.float32),
        scratch_types=[
            pltpu.VMEM((CHUNK,), jnp.int32),
            pltpu.VMEM((CHUNK, D), jnp.float32),
            pltpu.SemaphoreType.DMA,
        ],
    )
    def emb(idx_hbm, w_hbm, out_hbm, idx_v, rows_v, sem):
        wid = lax.axis_index("s") * NUM_CORES + lax.axis_index("c")
        base = wid * n_per_w

        @pl.loop(0, n_chunks)
        def _(ci):
            cb = base + ci * CHUNK
            pltpu.sync_copy(idx_hbm.at[pl.ds(part * n_part + cb, CHUNK)], idx_v)
            pltpu.async_copy(w_hbm.at[idx_v], rows_v, sem).wait()

            @pl.loop(0, CHUNK)
            def _(r):
                for c in range(0, D, 16):
                    sl = (r, pl.ds(c, 16))
                    rows_v.at[sl][...] = rows_v.at[sl][...] * scale

            pltpu.sync_copy(rows_v, out_hbm.at[pl.ds(cb, CHUNK)])

    return emb


def kernel(token_ids, W):
    B, S = token_ids.shape
    V, D = W.shape
    N = B * S
    scale = math.sqrt(D)
    n_part = N // PARTS
    b_part = B // PARTS
    assert n_part * PARTS == N and b_part * PARTS == B

    idx = token_ids.reshape(N).astype(jnp.int32)
    parts = []
    for p in range(PARTS):
        stage = _emb_call(n_part, V, D, scale, p)(idx, W)
        parts.append(stage.reshape(b_part, S, D))
    return jnp.concatenate(parts, axis=0)


# speedup vs baseline: 1.0411x; 1.0411x over previous
import functools
import math

import jax
import jax.numpy as jnp
from jax import lax
from jax.experimental import pallas as pl
from jax.experimental.pallas import tpu as pltpu
from jax.experimental.pallas import tpu_sc as plsc

NUM_CORES = 2
NUM_SUBCORES = 16
NUM_WORKERS = NUM_CORES * NUM_SUBCORES
CHUNK = 800
PARTS = 4


def _emb_call(n_part, V, D, scale, part):
    n_per_w = n_part // NUM_WORKERS
    n_chunks = n_per_w // CHUNK
    assert n_chunks * CHUNK * NUM_WORKERS == n_part
    assert n_chunks % 2 == 0
    mesh = plsc.VectorSubcoreMesh(core_axis_name="c", subcore_axis_name="s")

    @functools.partial(
        pl.kernel,
        mesh=mesh,
        compiler_params=pltpu.CompilerParams(use_tc_tiling_on_sc=False),
        out_type=jax.ShapeDtypeStruct((n_part, D), jnp.float32),
        scratch_types=[
            pltpu.VMEM((CHUNK,), jnp.int32),
            pltpu.VMEM((CHUNK,), jnp.int32),
            pltpu.VMEM((CHUNK, D), jnp.float32),
            pltpu.VMEM((CHUNK, D), jnp.float32),
            pltpu.SemaphoreType.DMA,
            pltpu.SemaphoreType.DMA,
            pltpu.SemaphoreType.DMA,
            pltpu.SemaphoreType.DMA,
        ],
    )
    def emb(idx_hbm, w_hbm, out_hbm, idx_a, idx_b, rows_a, rows_b,
            gsem_a, gsem_b, osem_a, osem_b):
        wid = lax.axis_index("s") * NUM_CORES + lax.axis_index("c")
        base = wid * n_per_w
        gbase = part * n_part + base

        def scale_rows(rows_v):
            @pl.loop(0, CHUNK)
            def _(r):
                for c in range(0, D, 16):
                    sl = (r, pl.ds(c, 16))
                    rows_v.at[sl][...] = rows_v.at[sl][...] * scale

        pltpu.sync_copy(idx_hbm.at[pl.ds(gbase, CHUNK)], idx_a)
        ga0 = pltpu.async_copy(w_hbm.at[idx_a], rows_a, gsem_a)

        @pl.loop(0, n_chunks, step=2)
        def _(ci):
            cb_a = base + ci * CHUNK
            cb_b = cb_a + CHUNK
            pltpu.sync_copy(idx_hbm.at[pl.ds(part * n_part + cb_b, CHUNK)],
                            idx_b)
            gb = pltpu.async_copy(w_hbm.at[idx_b], rows_b, gsem_b)
            pltpu.make_async_copy(w_hbm.at[idx_a], rows_a, gsem_a).wait()
            scale_rows(rows_a)
            oa = pltpu.async_copy(rows_a, out_hbm.at[pl.ds(cb_a, CHUNK)],
                                  osem_a)

            @pl.when(ci + 2 < n_chunks)
            def _():
                cb_n = base + (ci + 2) * CHUNK
                pltpu.sync_copy(
                    idx_hbm.at[pl.ds(part * n_part + cb_n, CHUNK)], idx_a)
                oa.wait()
                pltpu.async_copy(w_hbm.at[idx_a], rows_a, gsem_a)

            @pl.when(ci + 2 >= n_chunks)
            def _():
                oa.wait()

            gb.wait()
            scale_rows(rows_b)
            pltpu.async_copy(rows_b, out_hbm.at[pl.ds(cb_b, CHUNK)],
                             osem_b).wait()

    return emb


def kernel(token_ids, W):
    B, S = token_ids.shape
    V, D = W.shape
    N = B * S
    scale = math.sqrt(D)
    n_part = N // PARTS
    b_part = B // PARTS
    assert n_part * PARTS == N and b_part * PARTS == B

    idx = token_ids.reshape(N).astype(jnp.int32)
    parts = []
    for p in range(PARTS):
        stage = _emb_call(n_part, V, D, scale, p)(idx, W)
        parts.append(stage.reshape(b_part, S, D))
    return jnp.concatenate(parts, axis=0)


# speedup vs baseline: 1.1162x; 1.0722x over previous
import functools
import math

import jax
import jax.numpy as jnp
from jax import lax
from jax.experimental import pallas as pl
from jax.experimental.pallas import tpu as pltpu
from jax.experimental.pallas import tpu_sc as plsc

NUM_CORES = 2
NUM_SUBCORES = 16
NUM_WORKERS = NUM_CORES * NUM_SUBCORES
CHUNK = 800
PARTS = 4


def _emb_call(n_part, V, D, scale, part):
    n_per_w = n_part // NUM_WORKERS
    n_chunks = n_per_w // CHUNK
    assert n_chunks * CHUNK * NUM_WORKERS == n_part
    assert n_chunks % 2 == 0
    mesh = plsc.VectorSubcoreMesh(core_axis_name="c", subcore_axis_name="s")

    @functools.partial(
        pl.kernel,
        mesh=mesh,
        compiler_params=pltpu.CompilerParams(use_tc_tiling_on_sc=False),
        out_type=jax.ShapeDtypeStruct((n_part, 128), jnp.float32),
        scratch_types=[
            pltpu.VMEM((CHUNK,), jnp.int32),
            pltpu.VMEM((CHUNK,), jnp.int32),
            pltpu.VMEM((CHUNK, D), jnp.float32),
            pltpu.VMEM((CHUNK, D), jnp.float32),
            pltpu.SemaphoreType.DMA,
            pltpu.SemaphoreType.DMA,
            pltpu.SemaphoreType.DMA,
            pltpu.SemaphoreType.DMA,
        ],
    )
    def emb(idx_hbm, w_hbm, out_hbm, idx_a, idx_b, rows_a, rows_b,
            gsem_a, gsem_b, osem_a, osem_b):
        wid = lax.axis_index("s") * NUM_CORES + lax.axis_index("c")
        base = wid * n_per_w
        gbase = part * n_part + base

        def scale_rows(rows_v):
            @pl.loop(0, CHUNK)
            def _(r):
                for c in range(0, D, 16):
                    sl = (r, pl.ds(c, 16))
                    rows_v.at[sl][...] = rows_v.at[sl][...] * scale

        pltpu.sync_copy(idx_hbm.at[pl.ds(gbase, CHUNK)], idx_a)
        ga0 = pltpu.async_copy(w_hbm.at[idx_a], rows_a, gsem_a)

        @pl.loop(0, n_chunks, step=2)
        def _(ci):
            cb_a = base + ci * CHUNK
            cb_b = cb_a + CHUNK
            pltpu.sync_copy(idx_hbm.at[pl.ds(part * n_part + cb_b, CHUNK)],
                            idx_b)
            gb = pltpu.async_copy(w_hbm.at[idx_b], rows_b, gsem_b)
            pltpu.make_async_copy(w_hbm.at[idx_a], rows_a, gsem_a).wait()
            scale_rows(rows_a)
            oa = pltpu.async_copy(
                rows_a, out_hbm.at[pl.ds(cb_a, CHUNK), pl.ds(0, D)], osem_a)

            @pl.when(ci + 2 < n_chunks)
            def _():
                cb_n = base + (ci + 2) * CHUNK
                pltpu.sync_copy(
                    idx_hbm.at[pl.ds(part * n_part + cb_n, CHUNK)], idx_a)
                oa.wait()
                pltpu.async_copy(w_hbm.at[idx_a], rows_a, gsem_a)

            @pl.when(ci + 2 >= n_chunks)
            def _():
                oa.wait()

            gb.wait()
            scale_rows(rows_b)
            pltpu.async_copy(
                rows_b, out_hbm.at[pl.ds(cb_b, CHUNK), pl.ds(0, D)],
                osem_b).wait()

    return emb


def kernel(token_ids, W):
    B, S = token_ids.shape
    V, D = W.shape
    N = B * S
    scale = math.sqrt(D)
    n_part = N // PARTS
    b_part = B // PARTS
    assert n_part * PARTS == N and b_part * PARTS == B

    idx = token_ids.reshape(N).astype(jnp.int32)
    parts = []
    for p in range(PARTS):
        stage = _emb_call(n_part, V, D, scale, p)(idx, W)
        parts.append(stage[:, :D].reshape(b_part, S, D))
    return jnp.concatenate(parts, axis=0)
